# baseline (speedup 1.0000x reference)
import functools
import math

import jax
import jax.numpy as jnp
from jax import lax
from jax.experimental import pallas as pl
from jax.experimental.pallas import tpu as pltpu
from jax.experimental.pallas import tpu_sc as plsc

_OFFSET = 2
_D = 1024
_NC = 2
_NS = 16
_NW = _NC * _NS
_LANES = 16

_CH = 32


def _make_sc_embed(B_sc, B_out):
    b_per_w = B_sc // _NW
    nch = b_per_w // _CH

    mesh = plsc.VectorSubcoreMesh(
        core_axis_name="c", subcore_axis_name="s",
        num_cores=_NC, num_subcores=_NS)

    @functools.partial(
        pl.kernel,
        out_type=jax.ShapeDtypeStruct((B_out, _D), jnp.float32),
        mesh=mesh,
        scratch_types=[
            pltpu.VMEM((nch, _CH), jnp.int32),
            pltpu.VMEM((_CH, _D), jnp.float32),
            pltpu.VMEM((_CH, _D), jnp.float32),
            pltpu.SemaphoreType.DMA,
            pltpu.SemaphoreType.DMA,
        ],
    )
    def sc_embed(pos_hbm, table_hbm, out_hbm, idx2, buf0, buf1, g0, g1):
        wid = lax.axis_index("s") * _NC + lax.axis_index("c")
        base = wid * b_per_w
        bufs = (buf0, buf1)
        gsems = (g0, g1)

        pltpu.sync_copy(pos_hbm.at[wid], idx2)

        def add_off(i, carry):
            for j in range(_CH // _LANES):
                sl = pl.ds(j * _LANES, _LANES)
                idx2[i, sl] = idx2[i, sl] + _OFFSET
            return carry
        lax.fori_loop(0, nch, add_off, 0)

        def gather(c, b):
            pltpu.async_copy(table_hbm.at[idx2.at[c]], bufs[b], gsems[b])

        def gather_wait(c, b):
            pltpu.make_async_copy(
                table_hbm.at[idx2.at[c]], bufs[b], gsems[b]).wait()

        gather(0, 0)
        gather(1, 1)

        def chunk_body(g, carry):
            for b in range(2):
                c = g * 2 + b
                gather_wait(c, b)
                pltpu.sync_copy(bufs[b], out_hbm.at[pl.ds(base + c * _CH, _CH)])

                @pl.when(c + 2 < nch)
                def _nxt():
                    gather(c + 2, b)
            return carry
        lax.fori_loop(0, nch // 2, chunk_body, 0)

    return sc_embed


_HALF = _D // 2


def _tc_sin_body(pos_ref, freq_ref, out_ref):
    p = pos_ref[0]
    f = freq_ref[...]
    ang = p * f
    out_ref[:, :_HALF] = jnp.sin(ang)
    out_ref[:, _HALF:] = jnp.cos(ang)


def _tc_sin_rows(pos_f32, freqs):
    n = pos_f32.shape[0]
    r = 128
    grid = n // r
    return pl.pallas_call(
        _tc_sin_body,
        grid=(grid,),
        in_specs=[
            pl.BlockSpec((1, r, 1), lambda i: (i, 0, 0)),
            pl.BlockSpec((1, _HALF), lambda i: (0, 0)),
        ],
        out_specs=pl.BlockSpec((r, _D), lambda i: (i, 0)),
        out_shape=jax.ShapeDtypeStruct((n, _D), jnp.float32),
    )(pos_f32.reshape(grid, r, 1), freqs.reshape(1, _HALF))


def kernel(positions, weights):
    B = positions.size
    flat = positions.reshape(-1).astype(jnp.int32)
    b_sc = (B * 3 // 4) // (_NW * _CH) * (_NW * _CH)

    pos3 = flat[:b_sc].reshape(_NW, b_sc // (_NW * _CH), _CH)
    out = _make_sc_embed(b_sc, B)(pos3, weights)

    scale = math.log(10000.0) / (_HALF - 1)
    freqs = jnp.exp(jnp.arange(_HALF, dtype=jnp.float32) * -scale)
    pos_tc = flat[b_sc:].astype(jnp.float32) + float(_OFFSET)
    tc_rows = _tc_sin_rows(pos_tc, freqs)

    out = lax.dynamic_update_slice(out, tc_rows, (b_sc, 0))
    return out.reshape(*positions.shape, _D)


# speedup vs baseline: 1.5019x; 1.5019x over previous
import functools

import jax
import jax.numpy as jnp
from jax import lax
from jax.experimental import pallas as pl
from jax.experimental.pallas import tpu as pltpu
from jax.experimental.pallas import tpu_sc as plsc

_OFFSET = 2
_D = 1024
_NC = 2
_NS = 16
_NW = _NC * _NS
_LANES = 16

_CH = 32


def _make_sc_embed(B):
    b_per_w = B // _NW
    nch = b_per_w // _CH

    mesh = plsc.VectorSubcoreMesh(
        core_axis_name="c", subcore_axis_name="s",
        num_cores=_NC, num_subcores=_NS)

    @functools.partial(
        pl.kernel,
        out_type=jax.ShapeDtypeStruct((B, _D), jnp.float32),
        mesh=mesh,
        scratch_types=[
            pltpu.VMEM((nch, _CH), jnp.int32),
            pltpu.VMEM((_CH, _D), jnp.float32),
            pltpu.VMEM((_CH, _D), jnp.float32),
            pltpu.SemaphoreType.DMA,
            pltpu.SemaphoreType.DMA,
        ],
    )
    def sc_embed(pos_hbm, table_hbm, out_hbm, idx2, buf0, buf1, g0, g1):
        wid = lax.axis_index("c") * _NS + lax.axis_index("s")
        base = wid * b_per_w
        bufs = (buf0, buf1)
        gsems = (g0, g1)

        pltpu.sync_copy(pos_hbm.at[wid], idx2)

        def add_off(i, carry):
            for j in range(_CH // _LANES):
                sl = pl.ds(j * _LANES, _LANES)
                idx2[i, sl] = idx2[i, sl] + _OFFSET
            return carry
        lax.fori_loop(0, nch, add_off, 0)

        def gather(c, b):
            pltpu.async_copy(table_hbm.at[idx2.at[c]], bufs[b], gsems[b])

        def gather_wait(c, b):
            pltpu.make_async_copy(
                table_hbm.at[idx2.at[c]], bufs[b], gsems[b]).wait()

        gather(0, 0)
        gather(1, 1)

        def chunk_body(g, carry):
            for b in range(2):
                c = g * 2 + b
                gather_wait(c, b)
                pltpu.sync_copy(bufs[b], out_hbm.at[pl.ds(base + c * _CH, _CH)])

                @pl.when(c + 2 < nch)
                def _nxt():
                    gather(c + 2, b)
            return carry
        lax.fori_loop(0, nch // 2, chunk_body, 0)

    return sc_embed


def kernel(positions, weights):
    B = positions.size
    pos3 = positions.reshape(_NW, B // (_NW * _CH), _CH).astype(jnp.int32)
    out = _make_sc_embed(B)(pos3, weights)
    return out.reshape(*positions.shape, _D)
